# Initial kernel scaffold; baseline (speedup 1.0000x reference)
import jax
import jax.numpy as jnp
from jax.experimental import pallas as pl


def kernel(x, edge_index, W1_rel, b1_rel, W1_root, W2_rel, b2_rel, W2_root, W3, b3, W4, b4, W5, b5):
    raise NotImplementedError("write your pallas kernel here")


# speedup vs baseline: 7.2121x; 7.2121x over previous
import functools

import jax
import jax.numpy as jnp
from jax import lax
from jax.experimental import pallas as pl
from jax.experimental.pallas import tpu as pltpu
from jax.experimental.pallas import tpu_sc as plsc

NW = 32
TILES = 16
K = 80
ZROWS = 40


def _seg_sum_body(n_chunks, nb, x_hbm, ei_hbm, out_hbm,
                  agg_sp, srcb, dstb, rows, zb, sem):
    c = lax.axis_index("c")
    s = lax.axis_index("s")
    wid = c * TILES + s

    def zrow(i, carry):
        for j in range(8):
            zb[i, pl.ds(j * 16, 16)] = jnp.zeros((16,), jnp.float32)
        return carry
    lax.fori_loop(0, ZROWS, zrow, 0)
    n_rounds = (n_chunks + TILES - 1) // TILES
    for r in range(n_rounds):
        cs = s + r * TILES

        @pl.when(cs < n_chunks)
        def _():
            pltpu.sync_copy(zb, agg_sp.at[pl.ds(cs * ZROWS, ZROWS)])
    plsc.subcore_barrier()

    pltpu.sync_copy(ei_hbm.at[0, wid], srcb)
    pltpu.sync_copy(ei_hbm.at[1, wid], dstb)

    def body(b, carry):
        pltpu.async_copy(x_hbm.at[srcb.at[b]], rows, sem).wait()
        pltpu.sync_copy(rows, agg_sp.at[dstb.at[b]], add=True)
        return carry
    lax.fori_loop(0, nb, body, 0)
    plsc.subcore_barrier()

    for r in range(n_rounds):
        cs = s + r * TILES

        @pl.when(cs < n_chunks)
        def _():
            pltpu.sync_copy(agg_sp.at[pl.ds(cs * ZROWS, ZROWS)], zb)
            pltpu.sync_copy(zb, out_hbm.at[c, pl.ds(cs * ZROWS, ZROWS)])


def _make_seg_sum(n, cdim, e):
    nb = e // (NW * K)
    n_chunks = n // ZROWS
    mesh = plsc.VectorSubcoreMesh(core_axis_name="c", subcore_axis_name="s")
    return pl.kernel(
        functools.partial(_seg_sum_body, n_chunks, nb),
        mesh=mesh,
        out_type=jax.ShapeDtypeStruct((2, n, cdim), jnp.float32),
        scratch_types=[
            pltpu.VMEM_SHARED((n, cdim), jnp.float32),
            pltpu.VMEM((nb, K), jnp.int32),
            pltpu.VMEM((nb, K), jnp.int32),
            pltpu.VMEM((K, cdim), jnp.float32),
            pltpu.VMEM((ZROWS, cdim), jnp.float32),
            pltpu.SemaphoreType.DMA,
        ],
    )


def _layer1_tc(p0, p1, xb, wrelT, wrootT, b, o):
    agg = p0[...] + p1[...]
    y = (jnp.dot(agg, wrelT[...], preferred_element_type=jnp.float32)
         + jnp.dot(xb[...], wrootT[...], preferred_element_type=jnp.float32)
         + b[...])
    o[...] = jnp.maximum(y, 0.0)


def _layer2_mlp_tc(q0, q1, x1b, wrelT, wrootT, b2,
                   w3aT, w3bT, b3, w4T, b4, w5T, b5, o):
    agg = q0[...] + q1[...]
    x2 = jnp.maximum(
        jnp.dot(agg, wrelT[...], preferred_element_type=jnp.float32)
        + jnp.dot(x1b[...], wrootT[...], preferred_element_type=jnp.float32)
        + b2[...], 0.0)
    h = jnp.maximum(
        jnp.dot(x1b[...], w3aT[...], preferred_element_type=jnp.float32)
        + jnp.dot(x2, w3bT[...], preferred_element_type=jnp.float32)
        + b3[...], 0.0)
    h = jnp.maximum(
        jnp.dot(h, w4T[...], preferred_element_type=jnp.float32) + b4[...],
        0.0)
    o[...] = jnp.dot(h, w5T[...], preferred_element_type=jnp.float32) + b5[...]


def kernel(x, edge_index, W1_rel, b1_rel, W1_root, W2_rel, b2_rel, W2_root,
           W3, b3, W4, b4, W5, b5):
    n, cdim = x.shape
    e = edge_index.shape[1]
    nb = e // (NW * K)
    ei = edge_index.reshape(2, NW, nb, K)

    seg_sum = _make_seg_sum(n, cdim, e)

    blk = 1000
    grid = (n // blk,)
    row_spec = pl.BlockSpec((blk, cdim), lambda i: (i, 0))
    full = lambda shp: pl.BlockSpec(shp, lambda i: tuple(0 for _ in shp))

    p = seg_sum(x, ei)
    x1 = pl.pallas_call(
        _layer1_tc,
        grid=grid,
        in_specs=[row_spec, row_spec, row_spec,
                  full((cdim, cdim)), full((cdim, cdim)), full((1, cdim))],
        out_specs=row_spec,
        out_shape=jax.ShapeDtypeStruct((n, cdim), jnp.float32),
    )(p[0], p[1], x, W1_rel.T, W1_root.T, b1_rel.reshape(1, cdim))

    q = seg_sum(x1, ei)
    d3 = W3.shape[0]
    d4 = W4.shape[0]
    d5 = W5.shape[0]
    out = pl.pallas_call(
        _layer2_mlp_tc,
        grid=grid,
        in_specs=[row_spec, row_spec, row_spec,
                  full((cdim, cdim)), full((cdim, cdim)), full((1, cdim)),
                  full((cdim, d3)), full((cdim, d3)), full((1, d3)),
                  full((d3, d4)), full((1, d4)),
                  full((d4, d5)), full((1, d5))],
        out_specs=pl.BlockSpec((blk, d5), lambda i: (i, 0)),
        out_shape=jax.ShapeDtypeStruct((n, d5), jnp.float32),
    )(q[0], q[1], x1, W2_rel.T, W2_root.T, b2_rel.reshape(1, cdim),
      W3[:, :cdim].T, W3[:, cdim:].T, b3.reshape(1, d3),
      W4.T, b4.reshape(1, d4), W5.T, b5.reshape(1, d5))
    return out


# speedup vs baseline: 8.9560x; 1.2418x over previous
import functools

import jax
import jax.numpy as jnp
from jax import lax
from jax.experimental import pallas as pl
from jax.experimental.pallas import tpu as pltpu
from jax.experimental.pallas import tpu_sc as plsc

NW = 32
TILES = 16
K = 80
ZROWS = 40


def _seg_sum_body(n_chunks, nb, x_hbm, src_hbm, dst_hbm, out_hbm,
                  agg_sp, srcb, dstb, rows0, rows1, sem):
    c = lax.axis_index("c")
    s = lax.axis_index("s")
    wid = c * TILES + s

    def zrow(i, carry):
        for j in range(8):
            rows0[i, pl.ds(j * 16, 16)] = jnp.zeros((16,), jnp.float32)
        return carry
    lax.fori_loop(0, K, zrow, 0)
    n_rounds = (n_chunks + TILES - 1) // TILES
    for r in range(n_rounds):
        cs = s + r * TILES

        @pl.when(cs < n_chunks)
        def _():
            pltpu.sync_copy(rows0, agg_sp.at[pl.ds(cs * K, K)])
    plsc.subcore_barrier()

    pltpu.sync_copy(src_hbm.at[wid], srcb)
    pltpu.sync_copy(dst_hbm.at[wid], dstb)

    def sidx(b):
        return srcb.at[pl.ds(b * K, K)]

    pltpu.async_copy(x_hbm.at[sidx(0)], rows0, sem)

    def pair(i, carry):
        b = i * 2
        pltpu.make_async_copy(x_hbm.at[sidx(b)], rows0, sem).wait()

        @pl.when(b + 1 < nb)
        def _():
            pltpu.async_copy(x_hbm.at[sidx(b + 1)], rows1, sem)
        pltpu.sync_copy(rows0, agg_sp.at[dstb.at[b]], add=True)

        @pl.when(b + 1 < nb)
        def _():
            pltpu.make_async_copy(x_hbm.at[sidx(b + 1)], rows1, sem).wait()

            @pl.when(b + 2 < nb)
            def _():
                pltpu.async_copy(x_hbm.at[sidx(b + 2)], rows0, sem)
            pltpu.sync_copy(rows1, agg_sp.at[dstb.at[b + 1]], add=True)
        return carry
    lax.fori_loop(0, (nb + 1) // 2, pair, 0)
    plsc.subcore_barrier()

    for r in range(n_rounds):
        cs = s + r * TILES
        buf = rows0 if r % 2 == 0 else rows1

        @pl.when(cs < n_chunks)
        def _():
            pltpu.sync_copy(agg_sp.at[pl.ds(cs * K, K)], buf)
            pltpu.sync_copy(buf, out_hbm.at[c, pl.ds(cs * K, K)])


def _make_seg_sum(n, cdim, e):
    nb = e // (NW * K)
    n_chunks = n // K
    mesh = plsc.VectorSubcoreMesh(core_axis_name="c", subcore_axis_name="s")
    return pl.kernel(
        functools.partial(_seg_sum_body, n_chunks, nb),
        mesh=mesh,
        out_type=jax.ShapeDtypeStruct((2, n, cdim), jnp.float32),
        scratch_types=[
            pltpu.VMEM_SHARED((n, cdim), jnp.float32),
            pltpu.VMEM((nb * K,), jnp.int32),
            pltpu.VMEM((nb, K), jnp.int32),
            pltpu.VMEM((K, cdim), jnp.float32),
            pltpu.VMEM((K, cdim), jnp.float32),
            pltpu.SemaphoreType.DMA,
        ],
    )


def _layer1_tc(p0, p1, xb, wrelT, wrootT, b, o):
    agg = p0[...] + p1[...]
    y = (jnp.dot(agg, wrelT[...], preferred_element_type=jnp.float32)
         + jnp.dot(xb[...], wrootT[...], preferred_element_type=jnp.float32)
         + b[...])
    o[...] = jnp.maximum(y, 0.0)


def _layer2_mlp_tc(q0, q1, x1b, wrelT, wrootT, b2,
                   w3aT, w3bT, b3, w4T, b4, w5T, b5, o):
    agg = q0[...] + q1[...]
    x2 = jnp.maximum(
        jnp.dot(agg, wrelT[...], preferred_element_type=jnp.float32)
        + jnp.dot(x1b[...], wrootT[...], preferred_element_type=jnp.float32)
        + b2[...], 0.0)
    h = jnp.maximum(
        jnp.dot(x1b[...], w3aT[...], preferred_element_type=jnp.float32)
        + jnp.dot(x2, w3bT[...], preferred_element_type=jnp.float32)
        + b3[...], 0.0)
    h = jnp.maximum(
        jnp.dot(h, w4T[...], preferred_element_type=jnp.float32) + b4[...],
        0.0)
    o[...] = jnp.dot(h, w5T[...], preferred_element_type=jnp.float32) + b5[...]


def kernel(x, edge_index, W1_rel, b1_rel, W1_root, W2_rel, b2_rel, W2_root,
           W3, b3, W4, b4, W5, b5):
    n, cdim = x.shape
    e = edge_index.shape[1]
    nb = e // (NW * K)
    src2 = edge_index[0].reshape(NW, nb * K)
    dst3 = edge_index[1].reshape(NW, nb, K)

    seg_sum = _make_seg_sum(n, cdim, e)

    blk = 1000
    grid = (n // blk,)
    row_spec = pl.BlockSpec((blk, cdim), lambda i: (i, 0))
    full = lambda shp: pl.BlockSpec(shp, lambda i: tuple(0 for _ in shp))

    p = seg_sum(x, src2, dst3)
    x1 = pl.pallas_call(
        _layer1_tc,
        grid=grid,
        in_specs=[row_spec, row_spec, row_spec,
                  full((cdim, cdim)), full((cdim, cdim)), full((1, cdim))],
        out_specs=row_spec,
        out_shape=jax.ShapeDtypeStruct((n, cdim), jnp.float32),
    )(p[0], p[1], x, W1_rel.T, W1_root.T, b1_rel.reshape(1, cdim))

    q = seg_sum(x1, src2, dst3)
    d3 = W3.shape[0]
    d4 = W4.shape[0]
    d5 = W5.shape[0]
    out = pl.pallas_call(
        _layer2_mlp_tc,
        grid=grid,
        in_specs=[row_spec, row_spec, row_spec,
                  full((cdim, cdim)), full((cdim, cdim)), full((1, cdim)),
                  full((cdim, d3)), full((cdim, d3)), full((1, d3)),
                  full((d3, d4)), full((1, d4)),
                  full((d4, d5)), full((1, d5))],
        out_specs=pl.BlockSpec((blk, d5), lambda i: (i, 0)),
        out_shape=jax.ShapeDtypeStruct((n, d5), jnp.float32),
    )(q[0], q[1], x1, W2_rel.T, W2_root.T, b2_rel.reshape(1, cdim),
      W3[:, :cdim].T, W3[:, cdim:].T, b3.reshape(1, d3),
      W4.T, b4.reshape(1, d4), W5.T, b5.reshape(1, d5))
    return out


# speedup vs baseline: 12.5650x; 1.4030x over previous
import functools

import jax
import jax.numpy as jnp
from jax import lax
from jax.experimental import pallas as pl
from jax.experimental.pallas import tpu as pltpu
from jax.experimental.pallas import tpu_sc as plsc

NW = 32
TILES = 16
K = 80
SEG = 24
NSEG = 6


def _seg_sum_body(n_chunks, nb, x_hbm, src_hbm, dst_hbm, out_hbm,
                  agg_sp, dstb, srca, srcb, rows0, rows1, rows2,
                  sem_g, sem_s):
    c = lax.axis_index("c")
    s = lax.axis_index("s")
    wid = c * TILES + s

    def zrow(i, carry):
        for j in range(8):
            rows0[i, pl.ds(j * 16, 16)] = jnp.zeros((16,), jnp.float32)
        return carry
    lax.fori_loop(0, K, zrow, 0)
    n_rounds = (n_chunks + TILES - 1) // TILES
    for r in range(n_rounds):
        cs = s + r * TILES

        @pl.when(cs < n_chunks)
        def _():
            pltpu.sync_copy(rows0, agg_sp.at[pl.ds(cs * K, K)])
    plsc.subcore_barrier()

    pltpu.sync_copy(dst_hbm.at[wid], dstb)
    pltpu.sync_copy(src_hbm.at[wid, pl.ds(0, SEG * K)], srca)

    rowbufs = (rows0, rows1, rows2)
    srcbufs = (srca, srcb)

    def gather(idx_slice, buf):
        pltpu.async_copy(x_hbm.at[idx_slice], buf, sem_g)

    gather(srca.at[pl.ds(0, K)], rows0)
    gather(srca.at[pl.ds(K, K)], rows1)
    for k in range(NSEG):
        cur = srcbufs[k % 2]

        def seg_body(g, carry, k=k, cur=cur):
            for j in range(3):
                loc = g * 3 + j
                b = k * SEG + loc
                if j == 0 and k + 1 < NSEG:
                    @pl.when(g == 0)
                    def _():
                        pltpu.sync_copy(
                            src_hbm.at[wid, pl.ds((k + 1) * SEG * K, SEG * K)],
                            srcbufs[(k + 1) % 2])

                @pl.when(b < nb)
                def _():
                    pltpu.make_async_copy(
                        x_hbm.at[srca.at[pl.ds(0, K)]],
                        rowbufs[j], sem_g).wait()
                    pltpu.async_copy(rowbufs[j], agg_sp.at[dstb.at[b]],
                                     sem_s, add=True)

                @pl.when((b >= 1) & (b + 2 < nb))
                def _():
                    pltpu.make_async_copy(
                        rows0, agg_sp.at[dstb.at[0]], sem_s).wait()

                @pl.when(b + 2 < nb)
                def _():
                    nbuf = rowbufs[(j + 2) % 3]
                    nloc = loc + 2

                    @pl.when(nloc < SEG)
                    def _():
                        gather(cur.at[pl.ds(nloc * K, K)], nbuf)
                    if k + 1 < NSEG:
                        @pl.when(nloc >= SEG)
                        def _():
                            gather(srcbufs[(k + 1) % 2]
                                   .at[pl.ds((nloc - SEG) * K, K)], nbuf)
            return carry
        lax.fori_loop(0, SEG // 3, seg_body, 0)
    for _ in range(3):
        pltpu.make_async_copy(rows0, agg_sp.at[dstb.at[0]], sem_s).wait()
    plsc.subcore_barrier()

    for r in range(n_rounds):
        cs = s + r * TILES
        buf = rows0 if r % 2 == 0 else rows1

        @pl.when(cs < n_chunks)
        def _():
            pltpu.sync_copy(agg_sp.at[pl.ds(cs * K, K)], buf)
            pltpu.sync_copy(buf, out_hbm.at[c, pl.ds(cs * K, K)])


def _make_seg_sum(n, cdim, e):
    nb = e // (NW * K)
    n_chunks = n // K
    mesh = plsc.VectorSubcoreMesh(core_axis_name="c", subcore_axis_name="s")
    return pl.kernel(
        functools.partial(_seg_sum_body, n_chunks, nb),
        mesh=mesh,
        out_type=jax.ShapeDtypeStruct((2, n, cdim), jnp.float32),
        scratch_types=[
            pltpu.VMEM_SHARED((n, cdim), jnp.float32),
            pltpu.VMEM((nb, K), jnp.int32),
            pltpu.VMEM((SEG * K,), jnp.int32),
            pltpu.VMEM((SEG * K,), jnp.int32),
            pltpu.VMEM((K, cdim), jnp.float32),
            pltpu.VMEM((K, cdim), jnp.float32),
            pltpu.VMEM((K, cdim), jnp.float32),
            pltpu.SemaphoreType.DMA,
            pltpu.SemaphoreType.DMA,
        ],
    )


def _layer1_tc(p0, p1, xb, wrelT, wrootT, b, o):
    agg = p0[...] + p1[...]
    y = (jnp.dot(agg, wrelT[...], preferred_element_type=jnp.float32)
         + jnp.dot(xb[...], wrootT[...], preferred_element_type=jnp.float32)
         + b[...])
    o[...] = jnp.maximum(y, 0.0)


def _layer2_mlp_tc(q0, q1, x1b, wrelT, wrootT, b2,
                   w3aT, w3bT, b3, w4T, b4, w5T, b5, o):
    agg = q0[...] + q1[...]
    x2 = jnp.maximum(
        jnp.dot(agg, wrelT[...], preferred_element_type=jnp.float32)
        + jnp.dot(x1b[...], wrootT[...], preferred_element_type=jnp.float32)
        + b2[...], 0.0)
    h = jnp.maximum(
        jnp.dot(x1b[...], w3aT[...], preferred_element_type=jnp.float32)
        + jnp.dot(x2, w3bT[...], preferred_element_type=jnp.float32)
        + b3[...], 0.0)
    h = jnp.maximum(
        jnp.dot(h, w4T[...], preferred_element_type=jnp.float32) + b4[...],
        0.0)
    o[...] = jnp.dot(h, w5T[...], preferred_element_type=jnp.float32) + b5[...]


def kernel(x, edge_index, W1_rel, b1_rel, W1_root, W2_rel, b2_rel, W2_root,
           W3, b3, W4, b4, W5, b5):
    n, cdim = x.shape
    e = edge_index.shape[1]
    nb = e // (NW * K)
    src2 = jnp.pad(edge_index[0].reshape(NW, nb * K),
                   ((0, 0), (0, NSEG * SEG * K - nb * K)))
    dst3 = edge_index[1].reshape(NW, nb, K)

    seg_sum = _make_seg_sum(n, cdim, e)

    blk = 1000
    grid = (n // blk,)
    row_spec = pl.BlockSpec((blk, cdim), lambda i: (i, 0))
    full = lambda shp: pl.BlockSpec(shp, lambda i: tuple(0 for _ in shp))

    p = seg_sum(x, src2, dst3)
    x1 = pl.pallas_call(
        _layer1_tc,
        grid=grid,
        in_specs=[row_spec, row_spec, row_spec,
                  full((cdim, cdim)), full((cdim, cdim)), full((1, cdim))],
        out_specs=row_spec,
        out_shape=jax.ShapeDtypeStruct((n, cdim), jnp.float32),
    )(p[0], p[1], x, W1_rel.T, W1_root.T, b1_rel.reshape(1, cdim))

    q = seg_sum(x1, src2, dst3)
    d3 = W3.shape[0]
    d4 = W4.shape[0]
    d5 = W5.shape[0]
    out = pl.pallas_call(
        _layer2_mlp_tc,
        grid=grid,
        in_specs=[row_spec, row_spec, row_spec,
                  full((cdim, cdim)), full((cdim, cdim)), full((1, cdim)),
                  full((cdim, d3)), full((cdim, d3)), full((1, d3)),
                  full((d3, d4)), full((1, d4)),
                  full((d4, d5)), full((1, d5))],
        out_specs=pl.BlockSpec((blk, d5), lambda i: (i, 0)),
        out_shape=jax.ShapeDtypeStruct((n, d5), jnp.float32),
    )(q[0], q[1], x1, W2_rel.T, W2_root.T, b2_rel.reshape(1, cdim),
      W3[:, :cdim].T, W3[:, cdim:].T, b3.reshape(1, d3),
      W4.T, b4.reshape(1, d4), W5.T, b5.reshape(1, d5))
    return out


# speedup vs baseline: 13.2876x; 1.0575x over previous
import functools

import jax
import jax.numpy as jnp
from jax import lax
from jax.experimental import pallas as pl
from jax.experimental.pallas import tpu as pltpu
from jax.experimental.pallas import tpu_sc as plsc

NW = 32
TILES = 16
K = 80
SEG = 24
NSEG = 6


def _seg_sum_body(n_chunks, nb, x_hbm, src_hbm, dst_hbm, out_hbm,
                  agg_sp, dstb, srca, srcb, rows0, rows1, rows2,
                  sem_g, sem_s):
    c = lax.axis_index("c")
    s = lax.axis_index("s")
    wid = c * TILES + s

    def zrow(i, carry):
        for j in range(8):
            rows0[i, pl.ds(j * 16, 16)] = jnp.zeros((16,), jnp.float32)
        return carry
    lax.fori_loop(0, K, zrow, 0)
    n_rounds = (n_chunks + TILES - 1) // TILES
    for r in range(n_rounds):
        cs = s + r * TILES

        @pl.when(cs < n_chunks)
        def _():
            pltpu.sync_copy(rows0, agg_sp.at[pl.ds(cs * K, K)])
    plsc.subcore_barrier()

    pltpu.sync_copy(dst_hbm.at[wid], dstb)
    pltpu.sync_copy(src_hbm.at[wid, pl.ds(0, SEG * K)], srca)

    rowbufs = (rows0, rows1, rows2)
    srcbufs = (srca, srcb)

    def gather(idx_slice, buf):
        pltpu.async_copy(x_hbm.at[idx_slice], buf, sem_g)

    gather(srca.at[pl.ds(0, K)], rows0)
    gather(srca.at[pl.ds(K, K)], rows1)
    for k in range(NSEG):
        cur = srcbufs[k % 2]

        def seg_body(g, carry, k=k, cur=cur):
            for j in range(3):
                loc = g * 3 + j
                b = k * SEG + loc
                if j == 0 and k + 1 < NSEG:
                    @pl.when(g == 0)
                    def _():
                        pltpu.sync_copy(
                            src_hbm.at[wid, pl.ds((k + 1) * SEG * K, SEG * K)],
                            srcbufs[(k + 1) % 2])

                @pl.when(b < nb)
                def _():
                    pltpu.make_async_copy(
                        x_hbm.at[srca.at[pl.ds(0, K)]],
                        rowbufs[j], sem_g).wait()
                    pltpu.async_copy(rowbufs[j], agg_sp.at[dstb.at[b]],
                                     sem_s, add=True)

                @pl.when((b >= 1) & (b + 2 < nb))
                def _():
                    pltpu.make_async_copy(
                        rows0, agg_sp.at[dstb.at[0]], sem_s).wait()

                @pl.when(b + 2 < nb)
                def _():
                    nbuf = rowbufs[(j + 2) % 3]
                    nloc = loc + 2

                    @pl.when(nloc < SEG)
                    def _():
                        gather(cur.at[pl.ds(nloc * K, K)], nbuf)
                    if k + 1 < NSEG:
                        @pl.when(nloc >= SEG)
                        def _():
                            gather(srcbufs[(k + 1) % 2]
                                   .at[pl.ds((nloc - SEG) * K, K)], nbuf)
            return carry
        lax.fori_loop(0, SEG // 3, seg_body, 0)
    for _ in range(3):
        pltpu.make_async_copy(rows0, agg_sp.at[dstb.at[0]], sem_s).wait()
    plsc.subcore_barrier()

    for r in range(n_rounds):
        cs = s + r * TILES
        buf = rows0 if r % 2 == 0 else rows1

        @pl.when(cs < n_chunks)
        def _():
            pltpu.sync_copy(agg_sp.at[pl.ds(cs * K, K)], buf)
            pltpu.sync_copy(buf, out_hbm.at[c, pl.ds(cs * K, K)])


def _make_seg_sum(n, cdim, e):
    nb = e // (NW * K)
    n_chunks = n // K
    mesh = plsc.VectorSubcoreMesh(core_axis_name="c", subcore_axis_name="s")
    return pl.kernel(
        functools.partial(_seg_sum_body, n_chunks, nb),
        mesh=mesh,
        out_type=jax.ShapeDtypeStruct((2, n, cdim), jnp.float32),
        scratch_types=[
            pltpu.VMEM_SHARED((n, cdim), jnp.float32),
            pltpu.VMEM((nb, K), jnp.int32),
            pltpu.VMEM((SEG * K,), jnp.int32),
            pltpu.VMEM((SEG * K,), jnp.int32),
            pltpu.VMEM((K, cdim), jnp.float32),
            pltpu.VMEM((K, cdim), jnp.float32),
            pltpu.VMEM((K, cdim), jnp.float32),
            pltpu.SemaphoreType.DMA,
            pltpu.SemaphoreType.DMA,
        ],
    )


def _dotT(a, w):
    return lax.dot_general(a, w, (((1,), (1,)), ((), ())),
                           preferred_element_type=jnp.float32)


def _layer1_tc(p, xb, wrel, wroot, b, o):
    agg = p[0] + p[1]
    y = _dotT(agg, wrel[...]) + _dotT(xb[...], wroot[...]) + b[...]
    o[...] = jnp.maximum(y, 0.0)


def _layer2_mlp_tc(q, x1b, wrel, wroot, b2, w3, b3, w4, b4, w5, b5, o):
    cdim = x1b.shape[1]
    agg = q[0] + q[1]
    x1v = x1b[...]
    x2 = jnp.maximum(
        _dotT(agg, wrel[...]) + _dotT(x1v, wroot[...]) + b2[...], 0.0)
    w3v = w3[...]
    h = jnp.maximum(
        _dotT(x1v, w3v[:, :cdim]) + _dotT(x2, w3v[:, cdim:]) + b3[...], 0.0)
    h = jnp.maximum(_dotT(h, w4[...]) + b4[...], 0.0)
    o[...] = _dotT(h, w5[...]) + b5[...]


def kernel(x, edge_index, W1_rel, b1_rel, W1_root, W2_rel, b2_rel, W2_root,
           W3, b3, W4, b4, W5, b5):
    n, cdim = x.shape
    e = edge_index.shape[1]
    nb = e // (NW * K)
    src2 = jnp.pad(edge_index[0].reshape(NW, nb * K),
                   ((0, 0), (0, NSEG * SEG * K - nb * K)))
    dst3 = edge_index[1].reshape(NW, nb, K)

    seg_sum = _make_seg_sum(n, cdim, e)

    blk = 1000
    grid = (n // blk,)
    row_spec = pl.BlockSpec((blk, cdim), lambda i: (i, 0))
    pair_spec = pl.BlockSpec((2, blk, cdim), lambda i: (0, i, 0))
    full = lambda shp: pl.BlockSpec(shp, lambda i: tuple(0 for _ in shp))

    p = seg_sum(x, src2, dst3)
    x1 = pl.pallas_call(
        _layer1_tc,
        grid=grid,
        in_specs=[pair_spec, row_spec,
                  full((cdim, cdim)), full((cdim, cdim)), full((1, cdim))],
        out_specs=row_spec,
        out_shape=jax.ShapeDtypeStruct((n, cdim), jnp.float32),
    )(p, x, W1_rel, W1_root, b1_rel.reshape(1, cdim))

    q = seg_sum(x1, src2, dst3)
    d3 = W3.shape[0]
    d4 = W4.shape[0]
    d5 = W5.shape[0]
    out = pl.pallas_call(
        _layer2_mlp_tc,
        grid=grid,
        in_specs=[pair_spec, row_spec,
                  full((cdim, cdim)), full((cdim, cdim)), full((1, cdim)),
                  full((d3, 2 * cdim)), full((1, d3)),
                  full((d4, d3)), full((1, d4)),
                  full((d5, d4)), full((1, d5))],
        out_specs=pl.BlockSpec((blk, d5), lambda i: (i, 0)),
        out_shape=jax.ShapeDtypeStruct((n, d5), jnp.float32),
    )(q, x1, W2_rel, W2_root, b2_rel.reshape(1, cdim),
      W3, b3.reshape(1, d3), W4, b4.reshape(1, d4), W5, b5.reshape(1, d5))
    return out


# speedup vs baseline: 14.5132x; 1.0922x over previous
import functools

import jax
import jax.numpy as jnp
from jax import lax
from jax.experimental import pallas as pl
from jax.experimental.pallas import tpu as pltpu
from jax.experimental.pallas import tpu_sc as plsc

NW = 32
TILES = 16
K = 64
SEG = 24
NSEG = 7
SPAN_S = 9984
SPAN_B = 10112
NB_S = SPAN_S // K
NB_B = SPAN_B // K
N_SMALL = 28
LAST_REM = SPAN_B - (NSEG - 1) * SEG * K
CH = 40


def _seg_sum_body(n_chunks, x_hbm, ei_hbm, out_hbm,
                  agg_sp, dstb, srca, srcb, rows0, rows1, rows2, rows3,
                  sem_g, sem_s):
    c = lax.axis_index("c")
    s = lax.axis_index("s")
    wid = c * TILES + s
    nb = jnp.where(wid >= N_SMALL, NB_B, NB_S)
    off = SPAN_S * wid + (SPAN_B - SPAN_S) * jnp.maximum(wid - N_SMALL, 0)

    def zrow(i, carry):
        for j in range(8):
            rows0[i, pl.ds(j * 16, 16)] = jnp.zeros((16,), jnp.float32)
        return carry
    lax.fori_loop(0, CH, zrow, 0)
    n_rounds = (n_chunks + TILES - 1) // TILES
    for r in range(n_rounds):
        cs = s + r * TILES

        @pl.when(cs < n_chunks)
        def _():
            pltpu.sync_copy(rows0.at[pl.ds(0, CH)],
                            agg_sp.at[pl.ds(cs * CH, CH)])
    plsc.subcore_barrier()

    pltpu.sync_copy(ei_hbm.at[1, pl.ds(off, SPAN_B)], dstb)
    pltpu.sync_copy(ei_hbm.at[0, pl.ds(off, SEG * K)], srca)

    rowbufs = (rows0, rows1, rows2, rows3)
    srcbufs = (srca, srcb)

    def gather(idx_slice, buf):
        pltpu.async_copy(x_hbm.at[idx_slice], buf, sem_g)

    gather(srca.at[pl.ds(0, K)], rows0)
    gather(srca.at[pl.ds(K, K)], rows1)
    gather(srca.at[pl.ds(2 * K, K)], rows2)
    for k in range(NSEG):
        cur = srcbufs[k % 2]

        def seg_body(g, carry, k=k, cur=cur):
            for j in range(4):
                loc = g * 4 + j
                b = k * SEG + loc
                if j == 0 and k + 1 < NSEG:
                    rem = SEG * K if k + 1 < NSEG - 1 else LAST_REM

                    @pl.when(g == 0)
                    def _():
                        pltpu.sync_copy(
                            ei_hbm.at[0, pl.ds(off + (k + 1) * SEG * K, rem)],
                            srcbufs[(k + 1) % 2].at[pl.ds(0, rem)])

                @pl.when(b < nb)
                def _():
                    pltpu.make_async_copy(
                        x_hbm.at[srca.at[pl.ds(0, K)]],
                        rowbufs[j], sem_g).wait()
                    pltpu.async_copy(rowbufs[j],
                                     agg_sp.at[dstb.at[pl.ds(b * K, K)]],
                                     sem_s, add=True)

                @pl.when((b >= 1) & (b + 3 < nb))
                def _():
                    pltpu.make_async_copy(
                        rows0, agg_sp.at[dstb.at[pl.ds(0, K)]],
                        sem_s).wait()

                @pl.when(b + 3 < nb)
                def _():
                    nbuf = rowbufs[(j + 3) % 4]
                    nloc = loc + 3

                    @pl.when(nloc < SEG)
                    def _():
                        gather(cur.at[pl.ds(nloc * K, K)], nbuf)
                    if k + 1 < NSEG:
                        @pl.when(nloc >= SEG)
                        def _():
                            gather(srcbufs[(k + 1) % 2]
                                   .at[pl.ds((nloc - SEG) * K, K)], nbuf)
            return carry
        lax.fori_loop(0, SEG // 4, seg_body, 0)
    for _ in range(4):
        pltpu.make_async_copy(rows0, agg_sp.at[dstb.at[pl.ds(0, K)]],
                              sem_s).wait()
    plsc.subcore_barrier()

    for r in range(n_rounds):
        cs = s + r * TILES
        buf = rows0 if r % 2 == 0 else rows1

        @pl.when(cs < n_chunks)
        def _():
            pltpu.sync_copy(agg_sp.at[pl.ds(cs * CH, CH)],
                            buf.at[pl.ds(0, CH)])
            pltpu.sync_copy(buf.at[pl.ds(0, CH)],
                            out_hbm.at[c, pl.ds(cs * CH, CH)])


def _make_seg_sum(n, cdim):
    n_chunks = n // CH
    mesh = plsc.VectorSubcoreMesh(core_axis_name="c", subcore_axis_name="s")
    return pl.kernel(
        functools.partial(_seg_sum_body, n_chunks),
        mesh=mesh,
        out_type=jax.ShapeDtypeStruct((2, n, cdim), jnp.float32),
        scratch_types=[
            pltpu.VMEM_SHARED((n, cdim), jnp.float32),
            pltpu.VMEM((SPAN_B,), jnp.int32),
            pltpu.VMEM((SEG * K,), jnp.int32),
            pltpu.VMEM((SEG * K,), jnp.int32),
            pltpu.VMEM((K, cdim), jnp.float32),
            pltpu.VMEM((K, cdim), jnp.float32),
            pltpu.VMEM((K, cdim), jnp.float32),
            pltpu.VMEM((K, cdim), jnp.float32),
            pltpu.SemaphoreType.DMA,
            pltpu.SemaphoreType.DMA,
        ],
    )


def _dotT(a, w):
    return lax.dot_general(a, w, (((1,), (1,)), ((), ())),
                           preferred_element_type=jnp.float32)


def _layer1_tc(p, xb, wrel, wroot, b, o):
    agg = p[0] + p[1]
    y = _dotT(agg, wrel[...]) + _dotT(xb[...], wroot[...]) + b[...]
    o[...] = jnp.maximum(y, 0.0)


def _layer2_mlp_tc(q, x1b, wrel, wroot, b2, w3, b3, w4, b4, w5, b5, o):
    cdim = x1b.shape[1]
    agg = q[0] + q[1]
    x1v = x1b[...]
    x2 = jnp.maximum(
        _dotT(agg, wrel[...]) + _dotT(x1v, wroot[...]) + b2[...], 0.0)
    w3v = w3[...]
    h = jnp.maximum(
        _dotT(x1v, w3v[:, :cdim]) + _dotT(x2, w3v[:, cdim:]) + b3[...], 0.0)
    h = jnp.maximum(_dotT(h, w4[...]) + b4[...], 0.0)
    o[...] = _dotT(h, w5[...]) + b5[...]


def kernel(x, edge_index, W1_rel, b1_rel, W1_root, W2_rel, b2_rel, W2_root,
           W3, b3, W4, b4, W5, b5):
    n, cdim = x.shape

    seg_sum = _make_seg_sum(n, cdim)

    blk = 1000
    grid = (n // blk,)
    row_spec = pl.BlockSpec((blk, cdim), lambda i: (i, 0))
    pair_spec = pl.BlockSpec((2, blk, cdim), lambda i: (0, i, 0))
    full = lambda shp: pl.BlockSpec(shp, lambda i: tuple(0 for _ in shp))

    p = seg_sum(x, edge_index)
    x1 = pl.pallas_call(
        _layer1_tc,
        grid=grid,
        in_specs=[pair_spec, row_spec,
                  full((cdim, cdim)), full((cdim, cdim)), full((1, cdim))],
        out_specs=row_spec,
        out_shape=jax.ShapeDtypeStruct((n, cdim), jnp.float32),
    )(p, x, W1_rel, W1_root, b1_rel.reshape(1, cdim))

    q = seg_sum(x1, edge_index)
    d3 = W3.shape[0]
    d4 = W4.shape[0]
    d5 = W5.shape[0]
    out = pl.pallas_call(
        _layer2_mlp_tc,
        grid=grid,
        in_specs=[pair_spec, row_spec,
                  full((cdim, cdim)), full((cdim, cdim)), full((1, cdim)),
                  full((d3, 2 * cdim)), full((1, d3)),
                  full((d4, d3)), full((1, d4)),
                  full((d5, d4)), full((1, d5))],
        out_specs=pl.BlockSpec((blk, d5), lambda i: (i, 0)),
        out_shape=jax.ShapeDtypeStruct((n, d5), jnp.float32),
    )(q, x1, W2_rel, W2_root, b2_rel.reshape(1, cdim),
      W3, b3.reshape(1, d3), W4, b4.reshape(1, d4), W5, b5.reshape(1, d5))
    return out
